# baseline (speedup 1.0000x reference)
import jax
import jax.numpy as jnp
from jax import lax
from jax.experimental import pallas as pl
from jax.experimental.pallas import tpu as pltpu
from jax.experimental.pallas import tpu_sc as plsc

NC = 2
NS = 16
L = 16
B = 16384

N_HI = 688
N_LO = 336
HI_CORE = 1
CH_HI = (352, 336)
CH_LO = (336,)

_P5 = (0.029808765243552946, -0.12995719765850117, 0.2838231830655296,
       -0.48969909032090775, 0.9991664010110769, 1.1447097560674194e-05)


def _bandit_loss_body(score_hbm, idx_hbm, rew_hbm, out_hbm,
                      idx_v, vals_v, rew_v, out_v,
                      sem_i, sem_g0, sem_g1, sem_r, sem_o):
    cid = lax.axis_index("c")
    sid = lax.axis_index("s")
    sem_g = [sem_g0, sem_g1]
    k = [jnp.float32(v) for v in _P5]

    def run(chunks, base):
        n = sum(chunks)
        offs = [sum(chunks[:c]) for c in range(len(chunks))]
        cp_i = pltpu.async_copy(
            idx_hbm.at[pl.ds(base, n)], idx_v.at[pl.ds(0, n)], sem_i)
        cp_r = pltpu.async_copy(
            rew_hbm.at[pl.ds(base, n)], rew_v.at[pl.ds(0, n)], sem_r)
        cp_i.wait()
        cp_g = [
            pltpu.async_copy(
                score_hbm.at[idx_v.at[pl.ds(offs[c], chunks[c])]],
                vals_v.at[pl.ds(offs[c], chunks[c])],
                sem_g[c],
            )
            for c in range(len(chunks))
        ]
        cp_r.wait()

        def vreg_step(i, off):
            s = pl.ds(off + i * L, L)
            x = vals_v[s]
            r = rew_v[s]
            u = jnp.exp(-jnp.abs(x))
            p = ((((k[0] * u + k[1]) * u + k[2]) * u + k[3]) * u + k[4]) * u + k[5]
            m = jnp.minimum(x, jnp.float32(0.0))
            out_v[s] = (p - m) * r
            return off

        cp_o = []
        for c in range(len(chunks)):
            cp_g[c].wait()
            lax.fori_loop(0, chunks[c] // L, vreg_step, offs[c], unroll=4)
            cp_o.append(pltpu.async_copy(
                out_v.at[pl.ds(offs[c], chunks[c])],
                out_hbm.at[pl.ds(base + offs[c], chunks[c])],
                sem_o,
            ))
        for cp in cp_o:
            cp.wait()

    @pl.when(cid == HI_CORE)
    def _():
        run(CH_HI, sid * N_HI)

    @pl.when(cid != HI_CORE)
    def _():
        run(CH_LO, NS * N_HI + sid * N_LO)


@jax.jit
def _bandit_loss(score, idx, rewards):
    mesh = plsc.VectorSubcoreMesh(core_axis_name="c", subcore_axis_name="s")
    return pl.kernel(
        _bandit_loss_body,
        out_type=jax.ShapeDtypeStruct((B,), jnp.float32),
        mesh=mesh,
        scratch_types=[
            pltpu.VMEM((N_HI,), jnp.int32),
            pltpu.VMEM((N_HI,), jnp.float32),
            pltpu.VMEM((N_HI,), jnp.float32),
            pltpu.VMEM((N_HI,), jnp.float32),
        ] + [pltpu.SemaphoreType.DMA] * 5,
    )(score, idx, rewards)


def kernel(score, chosen_action, rewards):
    idx = chosen_action.astype(jnp.int32)
    return _bandit_loss(score, idx, rewards)


# speedup vs baseline: 1.0385x; 1.0385x over previous
import jax
import jax.numpy as jnp
from jax import lax
from jax.experimental import pallas as pl
from jax.experimental.pallas import tpu as pltpu
from jax.experimental.pallas import tpu_sc as plsc

NC = 2
NS = 16
L = 16
NW = NC * NS

B = 16384
PER_W = B // NW
CHUNK = 256
N_CHUNKS = PER_W // CHUNK

_P5 = (0.029808765243552946, -0.12995719765850117, 0.2838231830655296,
       -0.48969909032090775, 0.9991664010110769, 1.1447097560674194e-05)


def _bandit_loss_body(score_hbm, idx_hbm, rew_hbm, out_hbm,
                      idx_v, vals_v, rew_v, out_v,
                      sem_i, sem_g0, sem_g1, sem_r, sem_o):
    wid = lax.axis_index("s") * NC + lax.axis_index("c")
    base = wid * PER_W
    sem_g = [sem_g0, sem_g1]
    k = [jnp.float32(v) for v in _P5]

    cp_i = pltpu.async_copy(idx_hbm.at[pl.ds(base, PER_W)], idx_v, sem_i)
    cp_r = pltpu.async_copy(rew_hbm.at[pl.ds(base, PER_W)], rew_v, sem_r)
    cp_i.wait()
    cp_g = [
        pltpu.async_copy(
            score_hbm.at[idx_v.at[pl.ds(c * CHUNK, CHUNK)]],
            vals_v.at[pl.ds(c * CHUNK, CHUNK)],
            sem_g[c],
        )
        for c in range(N_CHUNKS)
    ]
    cp_r.wait()

    def vreg_step(i, off):
        s = pl.ds(off + i * L, L)
        x = vals_v[s]
        r = rew_v[s]
        u = jnp.exp(-jnp.abs(x))
        p = ((((k[0] * u + k[1]) * u + k[2]) * u + k[3]) * u + k[4]) * u + k[5]
        m = jnp.minimum(x, jnp.float32(0.0))
        out_v[s] = (p - m) * r
        return off

    cp_o = []
    for c in range(N_CHUNKS):
        cp_g[c].wait()
        lax.fori_loop(0, CHUNK // L, vreg_step, c * CHUNK, unroll=8)
        cp_o.append(pltpu.async_copy(
            out_v.at[pl.ds(c * CHUNK, CHUNK)],
            out_hbm.at[pl.ds(base + c * CHUNK, CHUNK)],
            sem_o,
        ))
    for cp in cp_o:
        cp.wait()


@jax.jit
def _bandit_loss(score, idx, rewards):
    mesh = plsc.VectorSubcoreMesh(core_axis_name="c", subcore_axis_name="s")
    return pl.kernel(
        _bandit_loss_body,
        out_type=jax.ShapeDtypeStruct((B,), jnp.float32),
        mesh=mesh,
        scratch_types=[
            pltpu.VMEM((PER_W,), jnp.int32),
            pltpu.VMEM((PER_W,), jnp.float32),
            pltpu.VMEM((PER_W,), jnp.float32),
            pltpu.VMEM((PER_W,), jnp.float32),
        ] + [pltpu.SemaphoreType.DMA] * 5,
    )(score, idx, rewards)


def kernel(score, chosen_action, rewards):
    idx = chosen_action.astype(jnp.int32)
    return _bandit_loss(score, idx, rewards)


# speedup vs baseline: 1.0700x; 1.0303x over previous
import jax
import jax.numpy as jnp
from jax import lax
from jax.experimental import pallas as pl
from jax.experimental.pallas import tpu as pltpu
from jax.experimental.pallas import tpu_sc as plsc

NC = 2
NS = 16
L = 16
NW = NC * NS

B = 16384
PER_W = B // NW
CHUNK = 256
N_CHUNKS = PER_W // CHUNK

_P5 = (0.029808765243552946, -0.12995719765850117, 0.2838231830655296,
       -0.48969909032090775, 0.9991664010110769, 1.1447097560674194e-05)


def _bandit_loss_body(score_hbm, idx_hbm, rew_hbm, out_hbm,
                      idx_v, vals_v, rew_v, out_v,
                      sem_i, sem_g0, sem_g1, sem_r, sem_o):
    wid = lax.axis_index("s") * NC + lax.axis_index("c")
    base = wid * PER_W
    sem_g = [sem_g0, sem_g1]

    cp_i = pltpu.async_copy(idx_hbm.at[pl.ds(base, PER_W)], idx_v, sem_i)
    cp_r = pltpu.async_copy(rew_hbm.at[pl.ds(base, PER_W)], rew_v, sem_r)
    cp_i.wait()
    cp_g = [
        pltpu.async_copy(
            score_hbm.at[idx_v.at[pl.ds(c * CHUNK, CHUNK)]],
            vals_v.at[pl.ds(c * CHUNK, CHUNK)],
            sem_g[c],
        )
        for c in range(N_CHUNKS)
    ]
    cp_r.wait()

    k = [jnp.float32(v) for v in _P5]
    cp_o = []
    for c in range(N_CHUNKS):
        cp_g[c].wait()
        for i in range(CHUNK // L):
            s = pl.ds(c * CHUNK + i * L, L)
            x = vals_v[s]
            r = rew_v[s]
            u = jnp.exp(-jnp.abs(x))
            p = ((((k[0] * u + k[1]) * u + k[2]) * u + k[3]) * u + k[4]) * u + k[5]
            m = jnp.minimum(x, jnp.float32(0.0))
            out_v[s] = (p - m) * r
        cp_o.append(pltpu.async_copy(
            out_v.at[pl.ds(c * CHUNK, CHUNK)],
            out_hbm.at[pl.ds(base + c * CHUNK, CHUNK)],
            sem_o,
        ))
    for cp in cp_o:
        cp.wait()


@jax.jit
def _bandit_loss(score, idx, rewards):
    mesh = plsc.VectorSubcoreMesh(core_axis_name="c", subcore_axis_name="s")
    return pl.kernel(
        _bandit_loss_body,
        out_type=jax.ShapeDtypeStruct((B,), jnp.float32),
        mesh=mesh,
        scratch_types=[
            pltpu.VMEM((PER_W,), jnp.int32),
            pltpu.VMEM((PER_W,), jnp.float32),
            pltpu.VMEM((PER_W,), jnp.float32),
            pltpu.VMEM((PER_W,), jnp.float32),
        ] + [pltpu.SemaphoreType.DMA] * 5,
    )(score, idx, rewards)


def kernel(score, chosen_action, rewards):
    idx = chosen_action.astype(jnp.int32)
    return _bandit_loss(score, idx, rewards)
